# Initial kernel scaffold; baseline (speedup 1.0000x reference)
import jax
import jax.numpy as jnp
from jax.experimental import pallas as pl


def kernel(input_ids, word_embeddings):
    raise NotImplementedError("write your pallas kernel here")


# speedup vs baseline: 1.6840x; 1.6840x over previous
import functools

import jax
import jax.numpy as jnp
from jax import lax
from jax.experimental import pallas as pl
from jax.experimental.pallas import tpu as pltpu
from jax.experimental.pallas import tpu_sc as plsc

NC = 2
NS = 16
NW = NC * NS
CH = 128


def _sc_gather(table, idx3, n_rows, d, nchunk):
    mesh = plsc.VectorSubcoreMesh(core_axis_name="c", subcore_axis_name="s")

    @functools.partial(
        pl.kernel,
        mesh=mesh,
        out_type=jax.ShapeDtypeStruct((n_rows, d), jnp.float32),
        scratch_types=[
            pltpu.VMEM((nchunk, CH), jnp.int32),
            pltpu.VMEM((CH, d), jnp.float32),
            pltpu.SemaphoreType.DMA,
        ],
        compiler_params=pltpu.CompilerParams(use_tc_tiling_on_sc=False),
    )
    def k(table_hbm, idx_hbm, out_hbm, idx_v, rows_v, sem):
        wid = lax.axis_index("s") * NC + lax.axis_index("c")
        pltpu.sync_copy(idx_hbm.at[wid], idx_v)
        base = wid * (nchunk * CH)

        def body(j, carry):
            pltpu.async_copy(table_hbm.at[idx_v.at[j]], rows_v, sem).wait()
            pltpu.sync_copy(rows_v, out_hbm.at[pl.ds(base + j * CH, CH)])
            return carry

        lax.fori_loop(0, nchunk, body, 0)

    return k(table, idx3)


def kernel(input_ids, word_embeddings):
    bsz, seq = input_ids.shape
    _, d = word_embeddings.shape
    n_rows = bsz * seq
    nchunk = n_rows // (NW * CH)
    idx3 = input_ids.reshape(NW, nchunk, CH).astype(jnp.int32)
    out = _sc_gather(word_embeddings, idx3, n_rows, d, nchunk)
    return out.reshape(bsz, seq, d)


# speedup vs baseline: 1.8798x; 1.1163x over previous
import functools

import jax
import jax.numpy as jnp
from jax import lax
from jax.experimental import pallas as pl
from jax.experimental.pallas import tpu as pltpu
from jax.experimental.pallas import tpu_sc as plsc

NC = 2
NS = 16
NW = NC * NS
CH = 128


K = 4


def _sc_gather(table, idx3, n_rows, d, nchunk):
    mesh = plsc.VectorSubcoreMesh(core_axis_name="c", subcore_axis_name="s")
    ngroups = nchunk // K

    @functools.partial(
        pl.kernel,
        mesh=mesh,
        out_type=jax.ShapeDtypeStruct((n_rows, d), jnp.float32),
        scratch_types=[
            pltpu.VMEM((nchunk, CH), jnp.int32),
            pltpu.VMEM((K, CH, d), jnp.float32),
            pltpu.VMEM((K, CH, d), jnp.float32),
            pltpu.SemaphoreType.DMA,
            pltpu.SemaphoreType.DMA,
        ],
        compiler_params=pltpu.CompilerParams(use_tc_tiling_on_sc=False),
    )
    def k(table_hbm, idx_hbm, out_hbm, idx_v, buf_a, buf_b, gsem, ssem):
        wid = lax.axis_index("s") * NC + lax.axis_index("c")
        pltpu.sync_copy(idx_hbm.at[wid], idx_v)
        base = wid * (nchunk * CH)

        def gather(j, buf, b):
            pltpu.async_copy(table_hbm.at[idx_v.at[j]], buf.at[b], gsem)

        def wait_gather(buf, b):
            pltpu.make_async_copy(table_hbm.at[idx_v.at[0]], buf.at[b], gsem).wait()

        def store(j, buf, b):
            pltpu.async_copy(buf.at[b], out_hbm.at[pl.ds(base + j * CH, CH)], ssem)

        def wait_store(buf, b):
            pltpu.make_async_copy(
                buf.at[b], out_hbm.at[pl.ds(base, CH)], ssem
            ).wait()

        for b in range(K):
            gather(b, buf_a, b)

        def half(g, cur, nxt):
            for b in range(K):
                wait_gather(cur, b)

            @pl.when(g + 1 < ngroups)
            def _():
                for b in range(K):
                    gather((g + 1) * K + b, nxt, b)

            for b in range(K):
                store(g * K + b, cur, b)
            for b in range(K):
                wait_store(cur, b)

        def body(t, carry):
            half(2 * t, buf_a, buf_b)
            half(2 * t + 1, buf_b, buf_a)
            return carry

        lax.fori_loop(0, ngroups // 2, body, 0)

    return k(table, idx3)


def kernel(input_ids, word_embeddings):
    bsz, seq = input_ids.shape
    _, d = word_embeddings.shape
    n_rows = bsz * seq
    nchunk = n_rows // (NW * CH)
    idx3 = input_ids.reshape(NW, nchunk, CH).astype(jnp.int32)
    out = _sc_gather(word_embeddings, idx3, n_rows, d, nchunk)
    return out.reshape(bsz, seq, d)
